# baseline (speedup 1.0000x reference)
import jax
import jax.numpy as jnp
from jax import lax
from jax.experimental import pallas as pl
from jax.experimental.pallas import tpu as pltpu
from jax.experimental.pallas import tpu_sc as plsc

N_NODES = 10000
N_EDGES = 320000
D = 128

NC = 2
NS = 16
L = 16

CHUNK = 64
NCH = 160
NR = NCH // 2
E_PAD = NC * NS * NCH * CHUNK
ROWS_TOT = E_PAD // (2 * CHUNK)
NBUF = 3

OB = 40
NB = N_NODES // OB
BPS = -(-NB // NS)


def _spmm_body(x_hbm, idx_hbm, w_hbm, part_hbm,
               idxs, ws, rows0, rows1, rows2,
               sb0, sb1, sb2, db0, db1, db2,
               g0, g1, g2, s0, s1, s2, acc):
    c = lax.axis_index("c")
    s = lax.axis_index("s")
    rows = (rows0, rows1, rows2)
    sbuf = (sb0, sb1, sb2)
    dbuf = (db0, db1, db2)
    gsem = (g0, g1, g2)
    ssem = (s0, s1, s2)

    r0 = (c * NS + s) * NR
    cp_idx = pltpu.async_copy(idx_hbm.at[pl.ds(r0, NR)], idxs, g0)
    cp_w = pltpu.async_copy(w_hbm.at[pl.ds(r0, NR)], ws, g1)

    zero = jnp.zeros((L,), jnp.float32)

    def zfill(i, _):
        for k in range(D // L):
            rows0[i, pl.ds(k * L, L)] = zero
        return 0

    lax.fori_loop(0, OB, zfill, 0)
    for k in range(BPS):
        b = s + k * NS

        @pl.when(b < NB)
        def _():
            pltpu.sync_copy(rows0.at[pl.ds(0, OB)], acc.at[pl.ds(b * OB, OB)])

    cp_idx.wait()
    cp_w.wait()
    plsc.subcore_barrier()

    def decode(j, b):
        r = j // 2
        c0 = (j % 2) * CHUNK
        for g in range(CHUNK // L):
            v = idxs[r, pl.ds(c0 + g * L, L)]
            sl = pl.ds(g * L, L)
            sbuf[b][sl] = v & 0xFFFF
            dbuf[b][sl] = lax.shift_right_logical(v, 16)

    def start_gather(j, b):
        decode(j, b)
        pltpu.async_copy(x_hbm.at[sbuf[b]], rows[b], gsem[b])

    def wait_gather(b):
        pltpu.make_async_copy(x_hbm.at[sbuf[b]], rows[b], gsem[b]).wait()

    def start_scatter(b):
        pltpu.async_copy(rows[b], acc.at[dbuf[b]], ssem[b], add=True)

    def wait_scatter(b):
        pltpu.make_async_copy(rows[b], acc.at[dbuf[b]], ssem[b]).wait()

    def scale(j, b):
        rb = rows[b]
        r = j // 2
        c0 = (j % 2) * CHUNK

        def mul(g, _):
            wv = ws[r, pl.ds(c0 + g * L, L)]
            for t in range(L):
                w = wv[t]
                i = g * L + t
                for k in range(D // L):
                    sl = pl.ds(k * L, L)
                    rb[i, sl] = rb[i, sl] * w
            return 0

        lax.fori_loop(0, CHUNK // L, mul, 0)

    start_gather(0, 0)
    start_gather(1, 1)
    wait_gather(0)
    scale(0, 0)
    start_scatter(0)
    start_gather(2, 2)

    def outer(k3, _):
        for u in range(NBUF):
            k = 3 * k3 + u + 1
            b = (u + 1) % NBUF
            wait_gather(b)
            scale(k, b)
            start_scatter(b)
            wait_scatter(u % NBUF)
            start_gather(k + 2, u % NBUF)
        return 0

    lax.fori_loop(0, (NCH - 4) // NBUF, outer, 0)

    t = NCH - 3
    wait_gather(t % NBUF)
    scale(t, t % NBUF)
    start_scatter(t % NBUF)
    wait_scatter((t + 2) % NBUF)
    start_gather(NCH - 1, (t + 2) % NBUF)
    wait_gather((t + 1) % NBUF)
    scale(t + 1, (t + 1) % NBUF)
    start_scatter((t + 1) % NBUF)
    wait_gather((t + 2) % NBUF)
    scale(t + 2, (t + 2) % NBUF)
    start_scatter((t + 2) % NBUF)

    for b in range(NBUF):
        wait_scatter(b)
    plsc.subcore_barrier()

    for k in range(BPS):
        b = s + k * NS

        @pl.when(b < NB)
        def _():
            r = b * OB
            pltpu.sync_copy(acc.at[pl.ds(r, OB)], rows0.at[pl.ds(0, OB)])
            pltpu.sync_copy(rows0.at[pl.ds(0, OB)], part_hbm.at[c, pl.ds(r, OB)])


_spmm = pl.kernel(
    _spmm_body,
    out_type=jax.ShapeDtypeStruct((NC, N_NODES, D), jnp.float32),
    mesh=plsc.VectorSubcoreMesh(core_axis_name="c", subcore_axis_name="s",
                                num_cores=NC, num_subcores=NS),
    scratch_types=[
        pltpu.VMEM((NR, 2 * CHUNK), jnp.int32),
        pltpu.VMEM((NR, 2 * CHUNK), jnp.float32),
        pltpu.VMEM((CHUNK, D), jnp.float32),
        pltpu.VMEM((CHUNK, D), jnp.float32),
        pltpu.VMEM((CHUNK, D), jnp.float32),
        pltpu.VMEM((CHUNK,), jnp.int32),
        pltpu.VMEM((CHUNK,), jnp.int32),
        pltpu.VMEM((CHUNK,), jnp.int32),
        pltpu.VMEM((CHUNK,), jnp.int32),
        pltpu.VMEM((CHUNK,), jnp.int32),
        pltpu.VMEM((CHUNK,), jnp.int32),
        pltpu.SemaphoreType.DMA,
        pltpu.SemaphoreType.DMA,
        pltpu.SemaphoreType.DMA,
        pltpu.SemaphoreType.DMA,
        pltpu.SemaphoreType.DMA,
        pltpu.SemaphoreType.DMA,
        pltpu.VMEM_SHARED((N_NODES, D), jnp.float32),
    ],
)


def _mm_body(x_ref, w_ref, o_ref):
    o_ref[...] = jnp.dot(x_ref[...], w_ref[...],
                         preferred_element_type=jnp.float32)


def _fuse_body(p_ref, w_ref, o_ref):
    h = jnp.maximum(p_ref[0] + p_ref[1], 0.0)
    o_ref[...] = jnp.dot(h, w_ref[...], preferred_element_type=jnp.float32)


def _add_body(q_ref, o_ref):
    o_ref[...] = q_ref[0] + q_ref[1]


_MB = 1000

_mm = pl.pallas_call(
    _mm_body,
    grid=(N_NODES // _MB,),
    in_specs=[pl.BlockSpec((_MB, D), lambda i: (i, 0)),
              pl.BlockSpec((D, D), lambda i: (0, 0))],
    out_specs=pl.BlockSpec((_MB, D), lambda i: (i, 0)),
    out_shape=jax.ShapeDtypeStruct((N_NODES, D), jnp.float32),
)

_fuse = pl.pallas_call(
    _fuse_body,
    grid=(N_NODES // _MB,),
    in_specs=[pl.BlockSpec((NC, _MB, D), lambda i: (0, i, 0)),
              pl.BlockSpec((D, D), lambda i: (0, 0))],
    out_specs=pl.BlockSpec((_MB, D), lambda i: (i, 0)),
    out_shape=jax.ShapeDtypeStruct((N_NODES, D), jnp.float32),
)

_add = pl.pallas_call(
    _add_body,
    grid=(N_NODES // _MB,),
    in_specs=[pl.BlockSpec((NC, _MB, D), lambda i: (0, i, 0))],
    out_specs=pl.BlockSpec((_MB, D), lambda i: (i, 0)),
    out_shape=jax.ShapeDtypeStruct((N_NODES, D), jnp.float32),
)


@jax.jit
def kernel(features, edge_index, edge_weight, W1, W2):
    pad = E_PAD - N_EDGES
    src = edge_index[0].astype(jnp.int32)
    dst = edge_index[1].astype(jnp.int32)
    packed = jnp.concatenate(
        [(dst << 16) | src, jnp.zeros((pad,), jnp.int32)]
    ).reshape(ROWS_TOT, 2 * CHUNK)
    w = jnp.concatenate(
        [edge_weight.astype(jnp.float32), jnp.zeros((pad,), jnp.float32)]
    ).reshape(ROWS_TOT, 2 * CHUNK)

    s1 = _mm(features, W1)
    p = _spmm(s1, packed, w)
    s2 = _fuse(p, W2)
    q = _spmm(s2, packed, w)
    return _add(q)
